# baseline (speedup 1.0000x reference)
import functools

import jax
import jax.numpy as jnp
from jax import lax
from jax.experimental import pallas as pl
from jax.experimental.pallas import tpu as pltpu
from jax.experimental.pallas import tpu_sc as plsc

D_MODEL = 128
T_LEN = 2048
N_OUTER = 256
N_WORKERS = 32
POS_PER_W = T_LEN // N_WORKERS
PE_CHUNK = POS_PER_W * D_MODEL
LANES = 16
UNROLL = 8
R_BLK = 8


def _sc_lookup_body(tab_hbm, pos_hbm, f_hbm, pe_hbm, idx_v, rows_v, fix_v, sem):
    wid = lax.axis_index("s") * 2 + lax.axis_index("c")

    pltpu.sync_copy(pos_hbm.at[wid], idx_v)
    pltpu.make_async_copy(tab_hbm.at[idx_v], rows_v, sem).start()
    pltpu.sync_copy(f_hbm.at[wid], fix_v)
    pltpu.make_async_copy(tab_hbm.at[idx_v], rows_v, sem).wait()

    def body(i, carry):
        for u in range(D_MODEL // LANES):
            off = u * LANES
            rows_v[i, pl.ds(off, LANES)] = (
                rows_v[i, pl.ds(off, LANES)] + fix_v[i, pl.ds(off, LANES)]
            )
        return carry
    lax.fori_loop(0, POS_PER_W, body, 0)

    pltpu.sync_copy(rows_v, pe_hbm.at[wid])


def _tc_body(x_ref, pe_ref, o_ref):
    o_ref[...] = x_ref[...] + pe_ref[...][None]


@jax.jit
def _run(x3, tab, pos, f3):
    mesh = plsc.VectorSubcoreMesh(core_axis_name="c", subcore_axis_name="s")
    sc_lookup = functools.partial(
        pl.kernel,
        mesh=mesh,
        out_type=jax.ShapeDtypeStruct((N_WORKERS, POS_PER_W, D_MODEL),
                                      jnp.float32),
        scratch_types=[
            pltpu.VMEM((POS_PER_W,), jnp.int32),
            pltpu.VMEM((POS_PER_W, D_MODEL), jnp.float32),
            pltpu.VMEM((POS_PER_W, D_MODEL), jnp.float32),
            pltpu.SemaphoreType.DMA,
        ],
    )(_sc_lookup_body)
    pe = sc_lookup(tab, pos, f3).reshape(T_LEN, D_MODEL)

    grid = (N_OUTER // R_BLK,)
    return pl.pallas_call(
        _tc_body,
        grid=grid,
        in_specs=[
            pl.BlockSpec((R_BLK, T_LEN, D_MODEL), lambda i: (i, 0, 0)),
            pl.BlockSpec((T_LEN, D_MODEL), lambda i: (0, 0)),
        ],
        out_specs=pl.BlockSpec((R_BLK, T_LEN, D_MODEL), lambda i: (i, 0, 0)),
        out_shape=jax.ShapeDtypeStruct((N_OUTER, T_LEN, D_MODEL), jnp.float32),
    )(x3, pe)


def kernel(x, learned_pe_table, fixed_pe):
    B, N, T, D = x.shape
    x3 = x.reshape(N_OUTER, T_LEN, D_MODEL)
    pos = jnp.arange(T_LEN, dtype=jnp.int32).reshape(N_WORKERS, POS_PER_W)
    f3 = fixed_pe.reshape(N_WORKERS, POS_PER_W, D_MODEL)
    out = _run(x3, learned_pe_table, pos, f3)
    return out.reshape(B, N, T, D)
